# Initial kernel scaffold; baseline (speedup 1.0000x reference)
import jax
import jax.numpy as jnp
from jax.experimental import pallas as pl


def kernel(x, edge_index, idx, proj_W, proj_b, gamma, beta, W0, b0, W1, b1):
    raise NotImplementedError("write your pallas kernel here")


# speedup vs baseline: 10.4213x; 10.4213x over previous
import functools

import jax
import jax.numpy as jnp
from jax import lax
from jax.experimental import pallas as pl
from jax.experimental.pallas import tpu as pltpu
from jax.experimental.pallas import tpu_sc as plsc

N = 10000
D = 128
NC = 2
NS = 16
NW = NC * NS
K = 128
BM = 1000
N_ACC = 10240
ROWS_PER_TILE = N_ACC // NS

_mesh = plsc.VectorSubcoreMesh(
    core_axis_name="c", subcore_axis_name="s", num_cores=NC, num_subcores=NS)


def _make_deg_kernel(chunks_per_tile):
  @functools.partial(
      pl.kernel,
      out_type=jax.ShapeDtypeStruct((NC * N_ACC, 16), jnp.float32),
      mesh=_mesh,
      scratch_types=[
          pltpu.VMEM((K,), jnp.int32),
          pltpu.VMEM((K, 16), jnp.float32),
          pltpu.VMEM((64, 16), jnp.float32),
          pltpu.VMEM_SHARED((N_ACC, 16), jnp.float32),
      ],
  )
  def deg_kernel(dst_hbm, out_hbm, dst_v, ones_v, zero_v, acc_sh):
    cid = lax.axis_index("c")
    sid = lax.axis_index("s")
    wid = cid * NS + sid

    def fill(i, _):
      ones_v[i, :] = jnp.ones((16,), jnp.float32)
      return 0
    lax.fori_loop(0, K, fill, 0)

    def zfill(i, _):
      zero_v[i, :] = jnp.zeros((16,), jnp.float32)
      return 0
    lax.fori_loop(0, 64, zfill, 0)

    for z in range(ROWS_PER_TILE // 64):
      pltpu.sync_copy(zero_v, acc_sh.at[pl.ds(sid * ROWS_PER_TILE + z * 64, 64)])
    plsc.subcore_barrier()

    def body(c, _):
      base = pl.multiple_of(wid * (chunks_per_tile * K) + c * K, K)
      pltpu.sync_copy(dst_hbm.at[pl.ds(base, K)], dst_v)
      pltpu.sync_copy(ones_v, acc_sh.at[dst_v], add=True)
      return 0
    lax.fori_loop(0, chunks_per_tile, body, 0)

    plsc.subcore_barrier()
    r0 = sid * ROWS_PER_TILE
    pltpu.sync_copy(acc_sh.at[pl.ds(r0, ROWS_PER_TILE)],
                    out_hbm.at[pl.ds(cid * N_ACC + r0, ROWS_PER_TILE)])

  return deg_kernel


def _make_spmm_kernel(chunks_per_tile):
  @functools.partial(
      pl.kernel,
      out_type=jax.ShapeDtypeStruct((NC * N_ACC, D), jnp.float32),
      mesh=_mesh,
      scratch_types=[
          pltpu.VMEM((K,), jnp.int32),
          pltpu.VMEM((K,), jnp.int32),
          pltpu.VMEM((K, D), jnp.float32),
          pltpu.VMEM((64, D), jnp.float32),
          pltpu.VMEM_SHARED((N_ACC, D), jnp.float32),
          pltpu.SemaphoreType.DMA,
      ],
  )
  def spmm_kernel(src_hbm, dst_hbm, g_hbm, out_hbm,
                  src_v, dst_v, rows_v, zero_v, acc_sh, sem):
    cid = lax.axis_index("c")
    sid = lax.axis_index("s")
    wid = cid * NS + sid

    def zfill(i, _):
      for j in range(D // 16):
        zero_v[i, pl.ds(j * 16, 16)] = jnp.zeros((16,), jnp.float32)
      return 0
    lax.fori_loop(0, 64, zfill, 0)

    for z in range(ROWS_PER_TILE // 64):
      pltpu.sync_copy(zero_v, acc_sh.at[pl.ds(sid * ROWS_PER_TILE + z * 64, 64)])
    plsc.subcore_barrier()

    def body(c, _):
      base = pl.multiple_of(wid * (chunks_per_tile * K) + c * K, K)
      pltpu.sync_copy(src_hbm.at[pl.ds(base, K)], src_v)
      pltpu.async_copy(g_hbm.at[src_v], rows_v, sem).wait()
      pltpu.sync_copy(dst_hbm.at[pl.ds(base, K)], dst_v)
      pltpu.sync_copy(rows_v, acc_sh.at[dst_v], add=True)
      return 0
    lax.fori_loop(0, chunks_per_tile, body, 0)

    plsc.subcore_barrier()
    r0 = sid * ROWS_PER_TILE
    pltpu.sync_copy(acc_sh.at[pl.ds(r0, ROWS_PER_TILE)],
                    out_hbm.at[pl.ds(cid * N_ACC + r0, ROWS_PER_TILE)])

  return spmm_kernel


def _make_gather_kernel(b):
  bpw = b // NW

  @functools.partial(
      pl.kernel,
      out_type=jax.ShapeDtypeStruct((b, D), jnp.float32),
      mesh=_mesh,
      scratch_types=[
          pltpu.VMEM((bpw,), jnp.int32),
          pltpu.VMEM((bpw, D), jnp.float32),
          pltpu.SemaphoreType.DMA,
      ],
  )
  def gather_kernel(table_hbm, idx_hbm, out_hbm, idx_v, rows_v, sem):
    cid = lax.axis_index("c")
    sid = lax.axis_index("s")
    wid = cid * NS + sid
    base = pl.multiple_of(wid * bpw, bpw)
    pltpu.sync_copy(idx_hbm.at[pl.ds(base, bpw)], idx_v)
    pltpu.async_copy(table_hbm.at[idx_v], rows_v, sem).wait()
    pltpu.sync_copy(rows_v, out_hbm.at[pl.ds(base, bpw)])

  return gather_kernel


def _proj_stats_body(x_ref, w_ref, b_ref, xw_ref, st_ref):
  i = pl.program_id(0)
  xw = jnp.dot(x_ref[...], w_ref[...], preferred_element_type=jnp.float32)
  xw = xw + b_ref[...]
  xw_ref[...] = xw

  @pl.when(i == 0)
  def _():
    st_ref[...] = jnp.zeros_like(st_ref)

  s = jnp.concatenate(
      [jnp.sum(xw, axis=0, keepdims=True),
       jnp.sum(xw * xw, axis=0, keepdims=True),
       jnp.zeros((6, D), jnp.float32)], axis=0)
  st_ref[...] += s


def _proj_stats(x, proj_w, proj_b):
  grid = N // BM
  return pl.pallas_call(
      _proj_stats_body,
      grid=(grid,),
      in_specs=[
          pl.BlockSpec((BM, D), lambda i: (i, 0)),
          pl.BlockSpec((D, D), lambda i: (0, 0)),
          pl.BlockSpec((1, D), lambda i: (0, 0)),
      ],
      out_specs=[
          pl.BlockSpec((BM, D), lambda i: (i, 0)),
          pl.BlockSpec((8, D), lambda i: (0, 0)),
      ],
      out_shape=[
          jax.ShapeDtypeStruct((N, D), jnp.float32),
          jax.ShapeDtypeStruct((8, D), jnp.float32),
      ],
  )(x, proj_w, proj_b)


def _g0_body(xw_ref, st_ref, d0_ref, d1_ref, w_ref, ga_ref, be_ref, g_ref):
  inv_n = 1.0 / N
  mean = st_ref[0:1] * inv_n
  var = st_ref[1:2] * inv_n - mean * mean
  scale = ga_ref[...] * lax.rsqrt(var + 1e-5)
  h = (xw_ref[...] - mean) * scale + be_ref[...]
  deg = d0_ref[0, :, 0:1] + d1_ref[0, :, 0:1] + 1.0
  dinv = lax.rsqrt(deg)
  g_ref[...] = dinv * jnp.dot(h, w_ref[...], preferred_element_type=jnp.float32)


def _g0(xw, stats, degp, w0, gamma, beta):
  grid = N // BM
  return pl.pallas_call(
      _g0_body,
      grid=(grid,),
      in_specs=[
          pl.BlockSpec((BM, D), lambda i: (i, 0)),
          pl.BlockSpec((8, D), lambda i: (0, 0)),
          pl.BlockSpec((1, BM, 16), lambda i: (0, i, 0)),
          pl.BlockSpec((1, BM, 16), lambda i: (1, i, 0)),
          pl.BlockSpec((D, D), lambda i: (0, 0)),
          pl.BlockSpec((1, D), lambda i: (0, 0)),
          pl.BlockSpec((1, D), lambda i: (0, 0)),
      ],
      out_specs=pl.BlockSpec((BM, D), lambda i: (i, 0)),
      out_shape=jax.ShapeDtypeStruct((N, D), jnp.float32),
  )(xw, stats, degp, degp, w0, gamma, beta)


def _g1_body(s0_ref, s1_ref, g_ref, d0_ref, d1_ref, w_ref, b_ref, out_ref):
  deg = d0_ref[0, :, 0:1] + d1_ref[0, :, 0:1] + 1.0
  dinv = lax.rsqrt(deg)
  h = dinv * (s0_ref[0] + s1_ref[0] + g_ref[...]) + b_ref[...]
  out_ref[...] = dinv * jnp.dot(h, w_ref[...], preferred_element_type=jnp.float32)


def _g1(sp, g0, degp, w1, b0):
  grid = N // BM
  return pl.pallas_call(
      _g1_body,
      grid=(grid,),
      in_specs=[
          pl.BlockSpec((1, BM, D), lambda i: (0, i, 0)),
          pl.BlockSpec((1, BM, D), lambda i: (1, i, 0)),
          pl.BlockSpec((BM, D), lambda i: (i, 0)),
          pl.BlockSpec((1, BM, 16), lambda i: (0, i, 0)),
          pl.BlockSpec((1, BM, 16), lambda i: (1, i, 0)),
          pl.BlockSpec((D, D), lambda i: (0, 0)),
          pl.BlockSpec((1, D), lambda i: (0, 0)),
      ],
      out_specs=pl.BlockSpec((BM, D), lambda i: (i, 0)),
      out_shape=jax.ShapeDtypeStruct((N, D), jnp.float32),
  )(sp, sp, g0, degp, degp, w1, b0)


def _h2_body(s0_ref, s1_ref, g_ref, d0_ref, d1_ref, b_ref, out_ref):
  deg = d0_ref[0, :, 0:1] + d1_ref[0, :, 0:1] + 1.0
  dinv = lax.rsqrt(deg)
  out_ref[...] = dinv * (s0_ref[0] + s1_ref[0] + g_ref[...]) + b_ref[...]


def _h2(sp, g1, degp, b1):
  grid = N // BM
  return pl.pallas_call(
      _h2_body,
      grid=(grid,),
      in_specs=[
          pl.BlockSpec((1, BM, D), lambda i: (0, i, 0)),
          pl.BlockSpec((1, BM, D), lambda i: (1, i, 0)),
          pl.BlockSpec((BM, D), lambda i: (i, 0)),
          pl.BlockSpec((1, BM, 16), lambda i: (0, i, 0)),
          pl.BlockSpec((1, BM, 16), lambda i: (1, i, 0)),
          pl.BlockSpec((1, D), lambda i: (0, 0)),
      ],
      out_specs=pl.BlockSpec((BM, D), lambda i: (i, 0)),
      out_shape=jax.ShapeDtypeStruct((N, D), jnp.float32),
  )(sp, sp, g1, degp, degp, b1)


def kernel(x, edge_index, idx, proj_W, proj_b, gamma, beta, W0, b0, W1, b1):
  e = edge_index.shape[1]
  per_tile = -(-e // (NW * K)) * K
  e_pad = per_tile * NW
  chunks_per_tile = per_tile // K

  src = jnp.concatenate(
      [edge_index[0], jnp.zeros((e_pad - e,), jnp.int32)])
  dst = jnp.concatenate(
      [edge_index[1], jnp.full((e_pad - e,), N, jnp.int32)])

  proj_b2 = proj_b.reshape(1, D)
  gamma2 = gamma.reshape(1, D)
  beta2 = beta.reshape(1, D)
  b0_2 = b0.reshape(1, D)
  b1_2 = b1.reshape(1, D)

  deg_kernel = _make_deg_kernel(chunks_per_tile)
  spmm_kernel = _make_spmm_kernel(chunks_per_tile)
  gather_kernel = _make_gather_kernel(idx.shape[0])

  degp = deg_kernel(dst).reshape(NC, N_ACC, 16)
  xw, stats = _proj_stats(x, proj_W, proj_b2)
  g0 = _g0(xw, stats, degp, W0, gamma2, beta2)
  s0 = spmm_kernel(src, dst, g0).reshape(NC, N_ACC, D)
  g1 = _g1(s0, g0, degp, W1, b0_2)
  s1 = spmm_kernel(src, dst, g1).reshape(NC, N_ACC, D)
  h2 = _h2(s1, g1, degp, b1_2)
  return gather_kernel(h2, idx)
